# baseline (speedup 1.0000x reference)
import functools

import jax
import jax.numpy as jnp
from jax import lax
from jax.experimental import pallas as pl
from jax.experimental.pallas import tpu as pltpu
from jax.experimental.pallas import tpu_sc as plsc

N_ROWS = 1048576
COL0 = 32
NC = 2
NS = 16
NW = NC * NS

TC_ROWS = 131072
SC_ROWS = N_ROWS - TC_ROWS
PER_W = SC_ROWS // NW
CHUNK = 2048
N_CHUNKS = PER_W // CHUNK
GROUPS = CHUNK // 16

TC_BLK = 16384

_mesh = plsc.VectorSubcoreMesh(core_axis_name="c", subcore_axis_name="s")


@functools.partial(
    pl.kernel,
    out_type=jax.ShapeDtypeStruct((NW, 16), jnp.float32),
    mesh=_mesh,
    scratch_types=[
        pltpu.VMEM((CHUNK, 16), jnp.float32),
        pltpu.VMEM((CHUNK, 16), jnp.float32),
        pltpu.VMEM((16,), jnp.float32),
        pltpu.SemaphoreType.DMA,
        pltpu.SemaphoreType.DMA,
    ],
    compiler_params=pltpu.CompilerParams(
        use_tc_tiling_on_sc=False, needs_layout_passes=False
    ),
)
def _sc_stage(ram_hbm, out_hbm, buf0, buf1, flag_v, sem0, sem1):
    wid = lax.axis_index("s") * NC + lax.axis_index("c")
    base = TC_ROWS + wid * PER_W

    bufs = (buf0, buf1)
    sems = (sem0, sem1)

    def fire(g):
        src = ram_hbm.at[pl.ds(base + g * CHUNK, CHUNK), pl.ds(COL0, 16)]
        return pltpu.async_copy(src, bufs[g % 2], sems[g % 2])

    lane = lax.iota(jnp.int32, 16)
    c0 = jnp.zeros((16,), jnp.int32)
    c1 = c0 + 1
    c2 = c0 + 2
    c3 = c0 + 3
    one = jnp.int32(1)

    bits = jnp.zeros((16,), jnp.int32)
    pending = fire(0)
    for g in range(N_CHUNKS):
        nxt = fire(g + 1) if g + 1 < N_CHUNKS else None
        pending.wait()
        buf = bufs[g % 2]

        def group(j, bits):
            row = lane + j * 16
            mi_x = plsc.load_gather(buf, [row, c0])
            su_x = plsc.load_gather(buf, [row, c1])
            mi_y = plsc.load_gather(buf, [row, c2])
            su_y = plsc.load_gather(buf, [row, c3])
            dist_x = jnp.abs(su_x - mi_x)
            dist_y = jnp.abs(su_y - mi_y)
            cond_y = dist_y > 4.0
            act_y = jnp.where(su_y < mi_y, 2, 5)
            targ = jnp.where(su_x < 80.0, su_x + 23.0, su_x - 23.0)
            dtx = mi_x - targ
            cl = dtx > 2.0
            cr = dtx < -2.0
            act_x = jnp.where(cl, 4, 3)
            cond_x = cl | cr
            punch = (dist_x <= 25.0) & (dist_y <= 8.0)
            action = jnp.where(cond_x, act_x, 0)
            action = jnp.where(cond_y, act_y, action)
            action = jnp.where(punch, 1, action)
            return bits | (one << action)

        bits = lax.fori_loop(0, GROUPS, group, bits)
        pending = nxt

    flags = jnp.zeros((16,), jnp.int32)
    for k in range(6):
        any_k = jnp.max((bits >> k) & 1)
        flags = jnp.where(lane == k, any_k, flags)
    flag_v[...] = flags.astype(jnp.float32)
    pltpu.sync_copy(flag_v, out_hbm.at[wid])


def _tc_stream_body(x_ref, o_ref):
    i = pl.program_id(0)
    x = x_ref[...]
    mi_x = x[:, COL0:COL0 + 1]
    su_x = x[:, COL0 + 1:COL0 + 2]
    mi_y = x[:, COL0 + 2:COL0 + 3]
    su_y = x[:, COL0 + 3:COL0 + 4]
    dist_x = jnp.abs(su_x - mi_x)
    dist_y = jnp.abs(su_y - mi_y)
    cond_y = dist_y > 4.0
    act_y = jnp.where(su_y < mi_y, 2, 5)
    targ = jnp.where(su_x < 80.0, su_x + 23.0, su_x - 23.0)
    dtx = mi_x - targ
    cl = dtx > 2.0
    cr = dtx < -2.0
    act_x = jnp.where(cl, 4, 3)
    cond_x = cl | cr
    punch = (dist_x <= 25.0) & (dist_y <= 8.0)
    action = jnp.where(cond_x, act_x, 0)
    action = jnp.where(cond_y, act_y, action)
    action = jnp.where(punch, 1, action)
    lanes = lax.broadcasted_iota(jnp.int32, (TC_BLK, 128), 1)
    onehot = (lanes == action).astype(jnp.float32)
    flags = jnp.max(onehot, axis=0, keepdims=True)

    @pl.when(i == 0)
    def _init():
        o_ref[...] = flags

    @pl.when(i > 0)
    def _acc():
        o_ref[...] = jnp.maximum(o_ref[...], flags)


def _tc_combine(sc_ref, tc_ref, o_ref):
    m = jnp.max(sc_ref[...], axis=0, keepdims=True)
    o_ref[...] = jnp.maximum(m[:, :6], tc_ref[:, :6])


def kernel(ram):
    sc_partial = _sc_stage(ram)
    tc_flags = pl.pallas_call(
        _tc_stream_body,
        grid=(TC_ROWS // TC_BLK,),
        in_specs=[pl.BlockSpec((TC_BLK, 128), lambda i: (i, 0))],
        out_specs=pl.BlockSpec((1, 128), lambda i: (0, 0)),
        out_shape=jax.ShapeDtypeStruct((1, 128), jnp.float32),
    )(ram[:TC_ROWS])
    return pl.pallas_call(
        _tc_combine,
        out_shape=jax.ShapeDtypeStruct((1, 6), jnp.float32),
    )(sc_partial, tc_flags)


# speedup vs baseline: 1.4131x; 1.4131x over previous
import functools

import jax
import jax.numpy as jnp
from jax import lax
from jax.experimental import pallas as pl
from jax.experimental.pallas import tpu as pltpu
from jax.experimental.pallas import tpu_sc as plsc

N_ROWS = 1048576
COL0 = 32
NC = 2
NS = 16
NW = NC * NS

TC_ROWS = 131072
SC_ROWS = N_ROWS - TC_ROWS
PER_W = SC_ROWS // NW
CHUNK = 2048
N_CHUNKS = PER_W // CHUNK
GROUPS = CHUNK // 16

TC_BLK = 16384

_mesh = plsc.VectorSubcoreMesh(core_axis_name="c", subcore_axis_name="s")


@functools.partial(
    pl.kernel,
    out_type=jax.ShapeDtypeStruct((NW, 16), jnp.float32),
    mesh=_mesh,
    scratch_types=[
        pltpu.VMEM((CHUNK, 16), jnp.float32),
        pltpu.VMEM((CHUNK, 16), jnp.float32),
        pltpu.VMEM((16,), jnp.float32),
        pltpu.SemaphoreType.DMA,
        pltpu.SemaphoreType.DMA,
    ],
    compiler_params=pltpu.CompilerParams(
        use_tc_tiling_on_sc=False, needs_layout_passes=False
    ),
)
def _sc_stage(ram_hbm, out_hbm, buf0, buf1, flag_v, sem0, sem1):
    wid = lax.axis_index("s") * NC + lax.axis_index("c")
    base = TC_ROWS + wid * PER_W

    bufs = (buf0, buf1)
    sems = (sem0, sem1)

    def fire(g):
        src = ram_hbm.at[pl.ds(base + g * CHUNK, CHUNK), pl.ds(COL0, 16)]
        return pltpu.async_copy(src, bufs[g % 2], sems[g % 2])

    lane = lax.iota(jnp.int32, 16)
    c0 = jnp.zeros((16,), jnp.int32)
    c1 = c0 + 1
    c2 = c0 + 2
    c3 = c0 + 3
    one = jnp.int32(1)

    bits = jnp.zeros((16,), jnp.int32)
    pending = fire(0)
    for g in range(N_CHUNKS):
        nxt = fire(g + 1) if g + 1 < N_CHUNKS else None
        pending.wait()
        buf = bufs[g % 2]

        def group(j, bits):
            row = lane + j * 16
            mi_x = plsc.load_gather(buf, [row, c0])
            su_x = plsc.load_gather(buf, [row, c1])
            mi_y = plsc.load_gather(buf, [row, c2])
            su_y = plsc.load_gather(buf, [row, c3])
            dist_x = jnp.abs(su_x - mi_x)
            dist_y = jnp.abs(su_y - mi_y)
            cond_y = dist_y > 4.0
            act_y = jnp.where(su_y < mi_y, 2, 5)
            targ = jnp.where(su_x < 80.0, su_x + 23.0, su_x - 23.0)
            dtx = mi_x - targ
            cl = dtx > 2.0
            cr = dtx < -2.0
            act_x = jnp.where(cl, 4, 3)
            cond_x = cl | cr
            punch = (dist_x <= 25.0) & (dist_y <= 8.0)
            action = jnp.where(cond_x, act_x, 0)
            action = jnp.where(cond_y, act_y, action)
            action = jnp.where(punch, 1, action)
            return bits | (one << action)

        bits = lax.fori_loop(0, GROUPS, group, bits)
        pending = nxt

    flags = jnp.zeros((16,), jnp.int32)
    for k in range(6):
        any_k = jnp.max((bits >> k) & 1)
        flags = jnp.where(lane == k, any_k, flags)
    flag_v[...] = flags.astype(jnp.float32)
    pltpu.sync_copy(flag_v, out_hbm.at[wid])


def _tc_stream_body(x_ref, o_ref):
    i = pl.program_id(0)
    x = x_ref[...]
    mi_x = x[:, COL0:COL0 + 1]
    su_x = x[:, COL0 + 1:COL0 + 2]
    mi_y = x[:, COL0 + 2:COL0 + 3]
    su_y = x[:, COL0 + 3:COL0 + 4]
    dist_x = jnp.abs(su_x - mi_x)
    dist_y = jnp.abs(su_y - mi_y)
    cond_y = dist_y > 4.0
    act_y = jnp.where(su_y < mi_y, 2, 5)
    targ = jnp.where(su_x < 80.0, su_x + 23.0, su_x - 23.0)
    dtx = mi_x - targ
    cl = dtx > 2.0
    cr = dtx < -2.0
    act_x = jnp.where(cl, 4, 3)
    cond_x = cl | cr
    punch = (dist_x <= 25.0) & (dist_y <= 8.0)
    action = jnp.where(cond_x, act_x, 0)
    action = jnp.where(cond_y, act_y, action)
    action = jnp.where(punch, 1, action)
    lanes = lax.broadcasted_iota(jnp.int32, (TC_BLK, 128), 1)
    onehot = (lanes == action).astype(jnp.float32)
    flags = jnp.max(onehot, axis=0, keepdims=True)

    @pl.when(i == 0)
    def _init():
        o_ref[...] = flags

    @pl.when(i > 0)
    def _acc():
        o_ref[...] = jnp.maximum(o_ref[...], flags)


def _tc_combine(sc_ref, tc_ref, o_ref):
    m = jnp.max(sc_ref[...], axis=0, keepdims=True)
    o_ref[...] = jnp.maximum(m[:, :6], tc_ref[:, :6])


def kernel(ram):
    sc_partial = _sc_stage(ram)
    tc_flags = pl.pallas_call(
        _tc_stream_body,
        grid=(TC_ROWS // TC_BLK,),
        in_specs=[pl.BlockSpec((TC_BLK, 128), lambda i: (i, 0))],
        out_specs=pl.BlockSpec((1, 128), lambda i: (0, 0)),
        out_shape=jax.ShapeDtypeStruct((1, 128), jnp.float32),
    )(ram)
    return pl.pallas_call(
        _tc_combine,
        out_shape=jax.ShapeDtypeStruct((1, 6), jnp.float32),
    )(sc_partial, tc_flags)


# speedup vs baseline: 2.7492x; 1.9455x over previous
import functools

import jax
import jax.numpy as jnp
from jax import lax
from jax.experimental import pallas as pl
from jax.experimental.pallas import tpu as pltpu
from jax.experimental.pallas import tpu_sc as plsc

N_ROWS = 1048576
N_COLS = 128
COL0 = 32
NC = 2
NS = 16
NW = NC * NS
PER_W = N_ROWS // NW
CHUNK = 2048
N_CHUNKS = PER_W // CHUNK
GROUPS = CHUNK // 16

_mesh = plsc.VectorSubcoreMesh(core_axis_name="c", subcore_axis_name="s")


@functools.partial(
    pl.kernel,
    out_type=jax.ShapeDtypeStruct((NW, 16), jnp.float32),
    mesh=_mesh,
    scratch_types=[
        pltpu.VMEM((CHUNK, 16), jnp.float32),
        pltpu.VMEM((CHUNK, 16), jnp.float32),
        pltpu.VMEM((16,), jnp.float32),
        pltpu.SemaphoreType.DMA,
        pltpu.SemaphoreType.DMA,
    ],
    compiler_params=pltpu.CompilerParams(
        use_tc_tiling_on_sc=False, needs_layout_passes=False
    ),
)
def _sc_stage1(ram_hbm, out_hbm, buf0, buf1, flag_v, sem0, sem1):
    wid = lax.axis_index("s") * NC + lax.axis_index("c")
    base = wid * PER_W

    bufs = (buf0, buf1)
    sems = (sem0, sem1)

    def fire(g):
        src = ram_hbm.at[pl.ds(base + g * CHUNK, CHUNK), pl.ds(COL0, 16)]
        return pltpu.async_copy(src, bufs[g % 2], sems[g % 2])

    lane = lax.iota(jnp.int32, 16)
    c0 = jnp.zeros((16,), jnp.int32)
    c1 = c0 + 1
    c2 = c0 + 2
    c3 = c0 + 3
    bits = jnp.zeros((16,), jnp.int32)
    pending = fire(0)
    for g in range(N_CHUNKS):
        nxt = fire(g + 1) if g + 1 < N_CHUNKS else None
        pending.wait()
        buf = bufs[g % 2]

        def group(j, bits):
            row = lane + j * 16
            mi_x = plsc.load_gather(buf, [row, c0])
            su_x = plsc.load_gather(buf, [row, c1])
            mi_y = plsc.load_gather(buf, [row, c2])
            su_y = plsc.load_gather(buf, [row, c3])
            dist_x = jnp.abs(su_x - mi_x)
            dist_y = jnp.abs(su_y - mi_y)
            cond_y = dist_y > 4.0
            b_y = jnp.where(su_y < mi_y, 4, 32)
            targ = su_x + jnp.where(su_x < 80.0, 23.0, -23.0)
            dtx = mi_x - targ
            cl = dtx > 2.0
            cr = dtx < -2.0
            b_x = jnp.where(cl, 16, 8)
            cond_x = cl | cr
            punch = (dist_x <= 25.0) & (dist_y <= 8.0)
            b = jnp.where(cond_x, b_x, 1)
            b = jnp.where(cond_y, b_y, b)
            b = jnp.where(punch, 2, b)
            return bits | b

        bits = lax.fori_loop(0, GROUPS, group, bits)
        pending = nxt

    flags = jnp.zeros((16,), jnp.int32)
    for k in range(6):
        any_k = jnp.max((bits >> k) & 1)
        flags = jnp.where(lane == k, any_k, flags)
    flag_v[...] = flags.astype(jnp.float32)
    pltpu.sync_copy(flag_v, out_hbm.at[wid])


def _tc_combine(p_ref, o_ref):
    m = jnp.max(p_ref[...], axis=0, keepdims=True)
    o_ref[...] = m[:, :6]


def kernel(ram):
    partial = _sc_stage1(ram)
    return pl.pallas_call(
        _tc_combine,
        out_shape=jax.ShapeDtypeStruct((1, 6), jnp.float32),
    )(partial)


# speedup vs baseline: 2.8921x; 1.0520x over previous
import functools

import jax
import jax.numpy as jnp
from jax import lax
from jax.experimental import pallas as pl
from jax.experimental.pallas import tpu as pltpu
from jax.experimental.pallas import tpu_sc as plsc

N_ROWS = 1048576
N_COLS = 128
COL0 = 32
NC = 2
NS = 16
NW = NC * NS
PER_W = N_ROWS // NW
CHUNK = 1024
N_CHUNKS = PER_W // CHUNK
GROUPS = CHUNK // 16

_mesh = plsc.VectorSubcoreMesh(core_axis_name="c", subcore_axis_name="s")


@functools.partial(
    pl.kernel,
    out_type=jax.ShapeDtypeStruct((NW, 16), jnp.float32),
    mesh=_mesh,
    scratch_types=[
        pltpu.VMEM((CHUNK, 16), jnp.float32),
        pltpu.VMEM((CHUNK, 16), jnp.float32),
        pltpu.VMEM((CHUNK, 16), jnp.float32),
        pltpu.VMEM((CHUNK, 16), jnp.float32),
        pltpu.VMEM((16,), jnp.float32),
        pltpu.SemaphoreType.DMA,
        pltpu.SemaphoreType.DMA,
        pltpu.SemaphoreType.DMA,
        pltpu.SemaphoreType.DMA,
    ],
    compiler_params=pltpu.CompilerParams(
        use_tc_tiling_on_sc=False, needs_layout_passes=False
    ),
)
def _sc_stage1(ram_hbm, out_hbm, buf0, buf1, buf2, buf3, flag_v, sem0, sem1, sem2, sem3):
    wid = lax.axis_index("s") * NC + lax.axis_index("c")
    base = wid * PER_W

    bufs = (buf0, buf1, buf2, buf3)
    sems = (sem0, sem1, sem2, sem3)
    NBUF = 4

    def fire(g):
        src = ram_hbm.at[pl.ds(base + g * CHUNK, CHUNK), pl.ds(COL0, 16)]
        return pltpu.async_copy(src, bufs[g % NBUF], sems[g % NBUF])

    lane = lax.iota(jnp.int32, 16)
    c0 = jnp.zeros((16,), jnp.int32)
    c1 = c0 + 1
    c2 = c0 + 2
    c3 = c0 + 3
    bits = jnp.zeros((16,), jnp.int32)
    inflight = [fire(g) for g in range(NBUF - 1)]
    for g in range(N_CHUNKS):
        if g + NBUF - 1 < N_CHUNKS:
            inflight.append(fire(g + NBUF - 1))
        inflight.pop(0).wait()
        buf = bufs[g % NBUF]

        def group(j, bits):
            row = lane + j * 16
            mi_x = plsc.load_gather(buf, [row, c0])
            su_x = plsc.load_gather(buf, [row, c1])
            mi_y = plsc.load_gather(buf, [row, c2])
            su_y = plsc.load_gather(buf, [row, c3])
            dist_x = jnp.abs(su_x - mi_x)
            dist_y = jnp.abs(su_y - mi_y)
            cond_y = dist_y > 4.0
            b_y = jnp.where(su_y < mi_y, 4, 32)
            targ = su_x + jnp.where(su_x < 80.0, 23.0, -23.0)
            dtx = mi_x - targ
            cl = dtx > 2.0
            cr = dtx < -2.0
            b_x = jnp.where(cl, 16, 8)
            cond_x = cl | cr
            punch = (dist_x <= 25.0) & (dist_y <= 8.0)
            b = jnp.where(cond_x, b_x, 1)
            b = jnp.where(cond_y, b_y, b)
            b = jnp.where(punch, 2, b)
            return bits | b

        bits = lax.fori_loop(0, GROUPS, group, bits)

    flags = jnp.zeros((16,), jnp.int32)
    for k in range(6):
        any_k = jnp.max((bits >> k) & 1)
        flags = jnp.where(lane == k, any_k, flags)
    flag_v[...] = flags.astype(jnp.float32)
    pltpu.sync_copy(flag_v, out_hbm.at[wid])


def _tc_combine(p_ref, o_ref):
    m = jnp.max(p_ref[...], axis=0, keepdims=True)
    o_ref[...] = m[:, :6]


def kernel(ram):
    partial = _sc_stage1(ram)
    return pl.pallas_call(
        _tc_combine,
        out_shape=jax.ShapeDtypeStruct((1, 6), jnp.float32),
    )(partial)


# speedup vs baseline: 2.9096x; 1.0060x over previous
import functools

import jax
import jax.numpy as jnp
from jax import lax
from jax.experimental import pallas as pl
from jax.experimental.pallas import tpu as pltpu
from jax.experimental.pallas import tpu_sc as plsc

N_ROWS = 1048576
N_COLS = 128
COL0 = 32
NC = 2
NS = 16
NW = NC * NS
PER_W = N_ROWS // NW
CHUNK = 1024
N_CHUNKS = PER_W // CHUNK
GROUPS = CHUNK // 16

_mesh = plsc.VectorSubcoreMesh(core_axis_name="c", subcore_axis_name="s")


@functools.partial(
    pl.kernel,
    out_type=jax.ShapeDtypeStruct((NW, 16), jnp.float32),
    mesh=_mesh,
    scratch_types=[
        pltpu.VMEM((CHUNK, 16), jnp.float32),
        pltpu.VMEM((CHUNK, 16), jnp.float32),
        pltpu.VMEM((CHUNK, 16), jnp.float32),
        pltpu.VMEM((CHUNK, 16), jnp.float32),
        pltpu.VMEM((CHUNK, 16), jnp.float32),
        pltpu.VMEM((CHUNK, 16), jnp.float32),
        pltpu.VMEM((16,), jnp.float32),
        pltpu.SemaphoreType.DMA,
        pltpu.SemaphoreType.DMA,
        pltpu.SemaphoreType.DMA,
        pltpu.SemaphoreType.DMA,
        pltpu.SemaphoreType.DMA,
        pltpu.SemaphoreType.DMA,
    ],
    compiler_params=pltpu.CompilerParams(
        use_tc_tiling_on_sc=False, needs_layout_passes=False
    ),
)
def _sc_stage1(ram_hbm, out_hbm, buf0, buf1, buf2, buf3, buf4, buf5, flag_v, sem0, sem1, sem2, sem3, sem4, sem5):
    wid = lax.axis_index("s") * NC + lax.axis_index("c")
    base = wid * PER_W

    bufs = (buf0, buf1, buf2, buf3, buf4, buf5)
    sems = (sem0, sem1, sem2, sem3, sem4, sem5)
    NBUF = 6

    def fire(g):
        src = ram_hbm.at[pl.ds(base + g * CHUNK, CHUNK), pl.ds(COL0, 16)]
        return pltpu.async_copy(src, bufs[g % NBUF], sems[g % NBUF])

    lane = lax.iota(jnp.int32, 16)
    c0 = jnp.zeros((16,), jnp.int32)
    c1 = c0 + 1
    c2 = c0 + 2
    c3 = c0 + 3
    bits = jnp.zeros((16,), jnp.int32)
    inflight = [fire(g) for g in range(NBUF - 1)]
    for g in range(N_CHUNKS):
        if g + NBUF - 1 < N_CHUNKS:
            inflight.append(fire(g + NBUF - 1))
        inflight.pop(0).wait()
        buf = bufs[g % NBUF]

        def group(j, bits):
            row = lane + j * 16
            mi_x = plsc.load_gather(buf, [row, c0])
            su_x = plsc.load_gather(buf, [row, c1])
            mi_y = plsc.load_gather(buf, [row, c2])
            su_y = plsc.load_gather(buf, [row, c3])
            dist_x = jnp.abs(su_x - mi_x)
            dist_y = jnp.abs(su_y - mi_y)
            cond_y = dist_y > 4.0
            b_y = jnp.where(su_y < mi_y, 4, 32)
            targ = su_x + jnp.where(su_x < 80.0, 23.0, -23.0)
            dtx = mi_x - targ
            cl = dtx > 2.0
            cr = dtx < -2.0
            b_x = jnp.where(cl, 16, 8)
            cond_x = cl | cr
            punch = (dist_x <= 25.0) & (dist_y <= 8.0)
            b = jnp.where(cond_x, b_x, 1)
            b = jnp.where(cond_y, b_y, b)
            b = jnp.where(punch, 2, b)
            return bits | b

        bits = lax.fori_loop(0, GROUPS, group, bits)

    flags = jnp.zeros((16,), jnp.int32)
    for k in range(6):
        any_k = jnp.max((bits >> k) & 1)
        flags = jnp.where(lane == k, any_k, flags)
    flag_v[...] = flags.astype(jnp.float32)
    pltpu.sync_copy(flag_v, out_hbm.at[wid])


def _tc_combine(p_ref, o_ref):
    m = jnp.max(p_ref[...], axis=0, keepdims=True)
    o_ref[...] = m[:, :6]


def kernel(ram):
    partial = _sc_stage1(ram)
    return pl.pallas_call(
        _tc_combine,
        out_shape=jax.ShapeDtypeStruct((1, 6), jnp.float32),
    )(partial)
